# Initial kernel scaffold; baseline (speedup 1.0000x reference)
import jax
import jax.numpy as jnp
from jax.experimental import pallas as pl


def kernel(nodes_features, edge_index, edge_type, basis_vector, rel_weight, weight_rel, loop_rel, w_in, w_out, w_loop):
    raise NotImplementedError("write your pallas kernel here")


# speedup vs baseline: 4.4923x; 4.4923x over previous
import jax
import jax.numpy as jnp
from jax import lax
from jax.experimental import pallas as pl
from jax.experimental.pallas import tpu as pltpu
from jax.experimental.pallas import tpu_sc as plsc

N = 10000
EH = 160000
D = 128
RP = 480
NBP = 56
LANES = 16
NSUB = 16
EPT = EH // NSUB
BLK = 80
NBLK = EPT // BLK
HB = BLK
ZCH = BLK
NCHUNK = N // ZCH
CPT = (NCHUNK + NSUB - 1) // NSUB

_sc_mesh = plsc.VectorSubcoreMesh(core_axis_name="c", subcore_axis_name="s", num_cores=1)


def _sc_body(heads_hbm, tails_hbm, types_hbm, nf_hbm, rel_hbm,
             acc_hbm, dinvw_hbm,
             head_v, tail_v, type_v, wblk_v, nfr_v, relr_v, outr_v,
             sem1, sem2, sem3, acc_sh, rel_sh):
    s = lax.axis_index("s")
    zero16 = jnp.zeros((LANES,), jnp.float32)
    one16 = jnp.ones((LANES,), jnp.float32)

    def fillz(r, _):
        for ch in range(D // LANES):
            outr_v[r, pl.ds(ch * LANES, LANES)] = zero16
        return 0

    def fillo(r, _):
        for ch in range(D // LANES):
            relr_v[r, pl.ds(ch * LANES, LANES)] = one16
        return 0

    lax.fori_loop(0, ZCH, fillz, 0)
    lax.fori_loop(0, HB, fillo, 0)

    @pl.when(s == 0)
    def _():
        pltpu.sync_copy(rel_hbm, rel_sh)

    for k in range(CPT):
        idx = s + NSUB * k

        @pl.when(idx < NCHUNK)
        def _(idx=idx):
            pltpu.sync_copy(outr_v, acc_sh.at[pl.ds(idx * ZCH, ZCH)])
    plsc.subcore_barrier()

    for d in (0, 1):
        def hblk(b, _, d=d):
            base = d * EH + s * EPT + b * HB
            pltpu.sync_copy(heads_hbm.at[pl.ds(base, HB)], head_v)
            pltpu.sync_copy(relr_v, acc_sh.at[head_v], add=True)
            return 0
        lax.fori_loop(0, EPT // HB, hblk, 0)
        plsc.subcore_barrier()

        for k in range(CPT):
            idx = s + NSUB * k

            @pl.when(idx < NCHUNK)
            def _(idx=idx, d=d):
                r0 = idx * ZCH
                pltpu.sync_copy(acc_sh.at[pl.ds(r0, ZCH)], nfr_v)

                def conv(r, _):
                    x = nfr_v[r, pl.ds(0, LANES)]
                    sq = x * (1.0 / 64.0) + 8.0
                    for _n in range(9):
                        sq = 0.5 * (sq + x / sq)
                    y = jnp.where(x > 0.5, 1.0 / sq, 0.0)
                    for ch in range(D // LANES):
                        nfr_v[r, pl.ds(ch * LANES, LANES)] = y
                    return 0
                lax.fori_loop(0, ZCH, conv, 0)
                pltpu.sync_copy(nfr_v, dinvw_hbm.at[pl.ds(d * N + r0, ZCH)])
                pltpu.sync_copy(outr_v, acc_sh.at[pl.ds(r0, ZCH)])
        plsc.subcore_barrier()

    for d in (0, 1):
        def blk(b, _, d=d):
            base = d * EH + s * EPT + b * BLK
            pltpu.sync_copy(heads_hbm.at[pl.ds(base, BLK)], head_v)
            pltpu.sync_copy(tails_hbm.at[pl.ds(base, BLK)], tail_v)
            pltpu.sync_copy(types_hbm.at[pl.ds(base, BLK)], type_v)
            cp1 = pltpu.async_copy(nf_hbm.at[tail_v], nfr_v, sem1)
            cp2 = pltpu.async_copy(rel_sh.at[type_v], relr_v, sem2)
            if d:
                for g in range(BLK // LANES):
                    sl = pl.ds(g * LANES, LANES)
                    tail_v[sl] = tail_v[sl] + N
            cp3 = pltpu.async_copy(dinvw_hbm.at[tail_v], wblk_v, sem3)
            cp1.wait()
            cp2.wait()
            cp3.wait()
            def ecomp(e, _):
                wj = wblk_v[e, pl.ds(0, LANES)]
                for ch in range(D // LANES):
                    sl = pl.ds(ch * LANES, LANES)
                    outr_v[e, sl] = wj * (nfr_v[e, sl] - relr_v[e, sl])
                return 0
            lax.fori_loop(0, BLK, ecomp, 0)
            pltpu.sync_copy(outr_v, acc_sh.at[head_v], add=True)
            return 0
        lax.fori_loop(0, NBLK, blk, 0)
        plsc.subcore_barrier()

        if d == 0:
            lax.fori_loop(0, ZCH, fillz, 0)

        for k in range(CPT):
            idx = s + NSUB * k

            @pl.when(idx < NCHUNK)
            def _(idx=idx, d=d):
                r0 = idx * ZCH
                pltpu.sync_copy(acc_sh.at[pl.ds(r0, ZCH)],
                                acc_hbm.at[pl.ds(d * N + r0, ZCH)])
                if d == 0:
                    pltpu.sync_copy(outr_v, acc_sh.at[pl.ds(r0, ZCH)])
        plsc.subcore_barrier()


def _sc_pass(heads_f, tails_f, types_f, nf, relp):
    k = pl.kernel(
        _sc_body,
        out_type=(jax.ShapeDtypeStruct((2 * N, D), jnp.float32),
                  jax.ShapeDtypeStruct((2 * N, D), jnp.float32)),
        mesh=_sc_mesh,
        scratch_types=[
            pltpu.VMEM((BLK,), jnp.int32),
            pltpu.VMEM((BLK,), jnp.int32),
            pltpu.VMEM((BLK,), jnp.int32),
            pltpu.VMEM((BLK, D), jnp.float32),
            pltpu.VMEM((BLK, D), jnp.float32),
            pltpu.VMEM((BLK, D), jnp.float32),
            pltpu.VMEM((BLK, D), jnp.float32),
            pltpu.SemaphoreType.DMA,
            pltpu.SemaphoreType.DMA,
            pltpu.SemaphoreType.DMA,
            pltpu.VMEM_SHARED((N, D), jnp.float32),
            pltpu.VMEM_SHARED((RP, D), jnp.float32),
        ],
    )
    return k(heads_f, tails_f, types_f, nf, relp)


def _rel_body(rw_ref, basis_ref, loop_ref, wr_ref, rel_ref, out2_ref):
    rel_main = jnp.dot(rw_ref[...], basis_ref[...],
                       preferred_element_type=jnp.float32)
    rows = lax.broadcasted_iota(jnp.int32, (RP, D), 0)
    rel = jnp.where(rows == 2 * 237, loop_ref[0:1, :], rel_main)
    rel_ref[...] = rel
    out2_ref[...] = jnp.dot(rel, wr_ref[...], preferred_element_type=jnp.float32)


def _tc_rel(rw_pad, basis_pad, loop8, weight_rel):
    return pl.pallas_call(
        _rel_body,
        out_shape=(jax.ShapeDtypeStruct((RP, D), jnp.float32),
                   jax.ShapeDtypeStruct((RP, D), jnp.float32)),
    )(rw_pad, basis_pad, loop8, weight_rel)


def _final_body(accin_ref, accout_ref, din_ref, dout_ref, nf_ref, loop_ref,
                win_ref, wout_ref, wloop_ref, out_ref):
    a = jnp.dot(din_ref[...] * accin_ref[...], win_ref[...],
                preferred_element_type=jnp.float32)
    a += jnp.dot(dout_ref[...] * accout_ref[...], wout_ref[...],
                 preferred_element_type=jnp.float32)
    a += jnp.dot(nf_ref[...] - loop_ref[0:1, :], wloop_ref[...],
                 preferred_element_type=jnp.float32)
    out_ref[...] = jnp.tanh(a * (1.0 / 3.0))


def _tc_final(acc_in, acc_out, dinv_in, dinv_out, nf, loop8, w_in, w_out, w_loop):
    BR = 1000
    grid = (N // BR,)
    row_bs = pl.BlockSpec((BR, D), lambda i: (i, 0))
    col_bs = pl.BlockSpec((BR, 1), lambda i: (i, 0))
    full_bs = pl.BlockSpec((D, D), lambda i: (0, 0))
    loop_bs = pl.BlockSpec((8, D), lambda i: (0, 0))
    return pl.pallas_call(
        _final_body,
        grid=grid,
        in_specs=[row_bs, row_bs, col_bs, col_bs, row_bs, loop_bs,
                  full_bs, full_bs, full_bs],
        out_specs=row_bs,
        out_shape=jax.ShapeDtypeStruct((N, D), jnp.float32),
    )(acc_in, acc_out, dinv_in, dinv_out, nf, loop8, w_in, w_out, w_loop)


def kernel(nodes_features, edge_index, edge_type, basis_vector, rel_weight,
           weight_rel, loop_rel, w_in, w_out, w_loop):
    nf = nodes_features
    heads_f = edge_index[0]
    tails_f = edge_index[1]
    types_f = edge_type

    rw_pad = jnp.pad(rel_weight, ((0, RP - 474), (0, NBP - 50)))
    basis_pad = jnp.pad(basis_vector, ((0, NBP - 50), (0, 0)))
    loop8 = jnp.broadcast_to(loop_rel, (8, D))
    relp, out2p = _tc_rel(rw_pad, basis_pad, loop8, weight_rel)
    out_2 = out2p[:475]

    acc, dinvw = _sc_pass(heads_f, tails_f, types_f, nf, relp)
    dinv2 = dinvw[:, 0].reshape(2, N)

    out = _tc_final(acc[:N], acc[N:], dinv2[0][:, None], dinv2[1][:, None],
                    nf, loop8, w_in, w_out, w_loop)
    return (out, out_2)


# speedup vs baseline: 8.4996x; 1.8920x over previous
import jax
import jax.numpy as jnp
from jax import lax
from jax.experimental import pallas as pl
from jax.experimental.pallas import tpu as pltpu
from jax.experimental.pallas import tpu_sc as plsc

N = 10000
EH = 160000
D = 128
RP = 480
NBP = 56
LANES = 16
NSUB = 16
EPT = EH // NSUB
BLK = 80
NBLK = EPT // BLK
HB = BLK
ZCH = BLK
NCHUNK = N // ZCH
CPT = (NCHUNK + NSUB - 1) // NSUB

_sc_mesh = plsc.VectorSubcoreMesh(core_axis_name="c", subcore_axis_name="s", num_cores=2)


def _sc_body(heads_hbm, tails_hbm, types_hbm, nf_hbm, rel_hbm,
             acc_hbm, dinvw_hbm,
             head_v, tail_v, type_v, wblk_v, nfr_v, relr_v, outr_v,
             sem1, sem2, sem3, acc_sh, rel_sh):
    s = lax.axis_index("s")
    c = lax.axis_index("c")
    zero16 = jnp.zeros((LANES,), jnp.float32)
    one16 = jnp.ones((LANES,), jnp.float32)

    def fillz(r, _):
        for ch in range(D // LANES):
            outr_v[r, pl.ds(ch * LANES, LANES)] = zero16
        return 0

    def fillo(r, _):
        for ch in range(D // LANES):
            relr_v[r, pl.ds(ch * LANES, LANES)] = one16
        return 0

    lax.fori_loop(0, ZCH, fillz, 0)
    lax.fori_loop(0, HB, fillo, 0)

    @pl.when(s == 0)
    def _():
        pltpu.sync_copy(rel_hbm, rel_sh)

    for k in range(CPT):
        idx = s + NSUB * k

        @pl.when(idx < NCHUNK)
        def _(idx=idx):
            pltpu.sync_copy(outr_v, acc_sh.at[pl.ds(idx * ZCH, ZCH)])
    plsc.subcore_barrier()

    def hblk(b, _):
        base = c * EH + s * EPT + b * HB
        pltpu.sync_copy(heads_hbm.at[pl.ds(base, HB)], head_v)
        pltpu.sync_copy(relr_v, acc_sh.at[head_v], add=True)
        return 0
    lax.fori_loop(0, EPT // HB, hblk, 0)
    plsc.subcore_barrier()

    for k in range(CPT):
        idx = s + NSUB * k

        @pl.when(idx < NCHUNK)
        def _(idx=idx):
            r0 = idx * ZCH
            pltpu.sync_copy(acc_sh.at[pl.ds(r0, ZCH)], nfr_v)

            def conv(r, _):
                x = nfr_v[r, pl.ds(0, LANES)]
                sq = x * (1.0 / 64.0) + 8.0
                for _n in range(9):
                    sq = 0.5 * (sq + x / sq)
                y = jnp.where(x > 0.5, 1.0 / sq, 0.0)
                for ch in range(D // LANES):
                    nfr_v[r, pl.ds(ch * LANES, LANES)] = y
                return 0
            lax.fori_loop(0, ZCH, conv, 0)
            pltpu.sync_copy(nfr_v, dinvw_hbm.at[pl.ds(c * N + r0, ZCH)])
            pltpu.sync_copy(outr_v, acc_sh.at[pl.ds(r0, ZCH)])
    plsc.subcore_barrier()

    def blk(b, _):
        base = c * EH + s * EPT + b * BLK
        pltpu.sync_copy(heads_hbm.at[pl.ds(base, BLK)], head_v)
        pltpu.sync_copy(tails_hbm.at[pl.ds(base, BLK)], tail_v)
        pltpu.sync_copy(types_hbm.at[pl.ds(base, BLK)], type_v)
        cp1 = pltpu.async_copy(nf_hbm.at[tail_v], nfr_v, sem1)
        cp2 = pltpu.async_copy(rel_sh.at[type_v], relr_v, sem2)
        off = c * N
        for g in range(BLK // LANES):
            sl = pl.ds(g * LANES, LANES)
            tail_v[sl] = tail_v[sl] + off
        cp3 = pltpu.async_copy(dinvw_hbm.at[tail_v], wblk_v, sem3)
        cp1.wait()
        cp2.wait()
        cp3.wait()
        def ecomp(e, _):
            wj = wblk_v[e, pl.ds(0, LANES)]
            for ch in range(D // LANES):
                sl = pl.ds(ch * LANES, LANES)
                outr_v[e, sl] = wj * (nfr_v[e, sl] - relr_v[e, sl])
            return 0
        lax.fori_loop(0, BLK, ecomp, 0)
        pltpu.sync_copy(outr_v, acc_sh.at[head_v], add=True)
        return 0
    lax.fori_loop(0, NBLK, blk, 0)
    plsc.subcore_barrier()

    for k in range(CPT):
        idx = s + NSUB * k

        @pl.when(idx < NCHUNK)
        def _(idx=idx):
            r0 = idx * ZCH
            pltpu.sync_copy(acc_sh.at[pl.ds(r0, ZCH)],
                            acc_hbm.at[pl.ds(c * N + r0, ZCH)])
    plsc.subcore_barrier()


def _sc_pass(heads_f, tails_f, types_f, nf, relp):
    k = pl.kernel(
        _sc_body,
        out_type=(jax.ShapeDtypeStruct((2 * N, D), jnp.float32),
                  jax.ShapeDtypeStruct((2 * N, D), jnp.float32)),
        mesh=_sc_mesh,
        scratch_types=[
            pltpu.VMEM((BLK,), jnp.int32),
            pltpu.VMEM((BLK,), jnp.int32),
            pltpu.VMEM((BLK,), jnp.int32),
            pltpu.VMEM((BLK, D), jnp.float32),
            pltpu.VMEM((BLK, D), jnp.float32),
            pltpu.VMEM((BLK, D), jnp.float32),
            pltpu.VMEM((BLK, D), jnp.float32),
            pltpu.SemaphoreType.DMA,
            pltpu.SemaphoreType.DMA,
            pltpu.SemaphoreType.DMA,
            pltpu.VMEM_SHARED((N, D), jnp.float32),
            pltpu.VMEM_SHARED((RP, D), jnp.float32),
        ],
    )
    return k(heads_f, tails_f, types_f, nf, relp)


def _rel_body(rw_ref, basis_ref, loop_ref, wr_ref, rel_ref, out2_ref):
    rel_main = jnp.dot(rw_ref[...], basis_ref[...],
                       preferred_element_type=jnp.float32)
    rows = lax.broadcasted_iota(jnp.int32, (RP, D), 0)
    rel = jnp.where(rows == 2 * 237, loop_ref[0:1, :], rel_main)
    rel_ref[...] = rel
    out2_ref[...] = jnp.dot(rel, wr_ref[...], preferred_element_type=jnp.float32)


def _tc_rel(rw_pad, basis_pad, loop8, weight_rel):
    return pl.pallas_call(
        _rel_body,
        out_shape=(jax.ShapeDtypeStruct((RP, D), jnp.float32),
                   jax.ShapeDtypeStruct((RP, D), jnp.float32)),
    )(rw_pad, basis_pad, loop8, weight_rel)


def _final_body(accin_ref, accout_ref, din_ref, dout_ref, nf_ref, loop_ref,
                win_ref, wout_ref, wloop_ref, out_ref):
    a = jnp.dot(din_ref[...] * accin_ref[...], win_ref[...],
                preferred_element_type=jnp.float32)
    a += jnp.dot(dout_ref[...] * accout_ref[...], wout_ref[...],
                 preferred_element_type=jnp.float32)
    a += jnp.dot(nf_ref[...] - loop_ref[0:1, :], wloop_ref[...],
                 preferred_element_type=jnp.float32)
    out_ref[...] = jnp.tanh(a * (1.0 / 3.0))


def _tc_final(acc_in, acc_out, dinv_in, dinv_out, nf, loop8, w_in, w_out, w_loop):
    BR = 1000
    grid = (N // BR,)
    row_bs = pl.BlockSpec((BR, D), lambda i: (i, 0))
    col_bs = pl.BlockSpec((BR, 1), lambda i: (i, 0))
    full_bs = pl.BlockSpec((D, D), lambda i: (0, 0))
    loop_bs = pl.BlockSpec((8, D), lambda i: (0, 0))
    return pl.pallas_call(
        _final_body,
        grid=grid,
        in_specs=[row_bs, row_bs, col_bs, col_bs, row_bs, loop_bs,
                  full_bs, full_bs, full_bs],
        out_specs=row_bs,
        out_shape=jax.ShapeDtypeStruct((N, D), jnp.float32),
    )(acc_in, acc_out, dinv_in, dinv_out, nf, loop8, w_in, w_out, w_loop)


def kernel(nodes_features, edge_index, edge_type, basis_vector, rel_weight,
           weight_rel, loop_rel, w_in, w_out, w_loop):
    nf = nodes_features
    heads_f = edge_index[0]
    tails_f = edge_index[1]
    types_f = edge_type

    rw_pad = jnp.pad(rel_weight, ((0, RP - 474), (0, NBP - 50)))
    basis_pad = jnp.pad(basis_vector, ((0, NBP - 50), (0, 0)))
    loop8 = jnp.broadcast_to(loop_rel, (8, D))
    relp, out2p = _tc_rel(rw_pad, basis_pad, loop8, weight_rel)
    out_2 = out2p[:475]

    acc, dinvw = _sc_pass(heads_f, tails_f, types_f, nf, relp)
    dinv2 = dinvw[:, 0].reshape(2, N)

    out = _tc_final(acc[:N], acc[N:], dinv2[0][:, None], dinv2[1][:, None],
                    nf, loop8, w_in, w_out, w_loop)
    return (out, out_2)


# speedup vs baseline: 9.8091x; 1.1541x over previous
import jax
import jax.numpy as jnp
from jax import lax
from jax.experimental import pallas as pl
from jax.experimental.pallas import tpu as pltpu
from jax.experimental.pallas import tpu_sc as plsc

N = 10000
EH = 160000
D = 128
RP = 480
NBP = 56
LANES = 16
NSUB = 16
EPT = EH // NSUB
BLK = 80
NBLK = EPT // BLK
HB = BLK
ZCH = BLK
NCHUNK = N // ZCH
CPT = (NCHUNK + NSUB - 1) // NSUB

_sc_mesh = plsc.VectorSubcoreMesh(core_axis_name="c", subcore_axis_name="s", num_cores=2)


def _sc_body(heads_hbm, tails_hbm, types_hbm, nf_hbm, rel_hbm,
             acc_hbm, dinvw_hbm,
             head_v, tail_v, type_v, wblk_v, nfr_v, relr_v, outr_v,
             sem1, sem2, sem3, acc_sh, rel_sh):
    s = lax.axis_index("s")
    c = lax.axis_index("c")
    zero16 = jnp.zeros((LANES,), jnp.float32)
    one16 = jnp.ones((LANES,), jnp.float32)

    def fillz(r, _):
        for ch in range(D // LANES):
            outr_v[r, pl.ds(ch * LANES, LANES)] = zero16
        return 0

    def fillo(r, _):
        for ch in range(D // LANES):
            relr_v[r, pl.ds(ch * LANES, LANES)] = one16
        return 0

    lax.fori_loop(0, ZCH, fillz, 0)
    lax.fori_loop(0, HB, fillo, 0)

    @pl.when(s == 0)
    def _():
        pltpu.sync_copy(rel_hbm, rel_sh)

    for k in range(CPT):
        idx = s + NSUB * k

        @pl.when(idx < NCHUNK)
        def _(idx=idx):
            pltpu.sync_copy(outr_v, acc_sh.at[pl.ds(idx * ZCH, ZCH)])
    plsc.subcore_barrier()

    def hblk(b, _):
        base = c * EH + s * EPT + b * HB
        pltpu.sync_copy(heads_hbm.at[pl.ds(base, HB)], head_v)
        pltpu.sync_copy(relr_v, acc_sh.at[head_v], add=True)
        return 0
    lax.fori_loop(0, EPT // HB, hblk, 0)
    plsc.subcore_barrier()

    for k in range(CPT):
        idx = s + NSUB * k

        @pl.when(idx < NCHUNK)
        def _(idx=idx):
            r0 = idx * ZCH
            pltpu.sync_copy(acc_sh.at[pl.ds(r0, ZCH)], nfr_v)

            def conv(r, _):
                x = nfr_v[r, pl.ds(0, LANES)]
                sq = x * (1.0 / 64.0) + 8.0
                for _n in range(9):
                    sq = 0.5 * (sq + x / sq)
                y = jnp.where(x > 0.5, 1.0 / sq, 0.0)
                for ch in range(D // LANES):
                    nfr_v[r, pl.ds(ch * LANES, LANES)] = y
                return 0
            lax.fori_loop(0, ZCH, conv, 0)
            pltpu.sync_copy(nfr_v, dinvw_hbm.at[pl.ds(c * N + r0, ZCH)])
            pltpu.sync_copy(outr_v, acc_sh.at[pl.ds(r0, ZCH)])
    plsc.subcore_barrier()

    def blk(b, _):
        base = c * EH + s * EPT + b * BLK
        ci1 = pltpu.async_copy(heads_hbm.at[pl.ds(base, BLK)], head_v, sem1)
        ci2 = pltpu.async_copy(tails_hbm.at[pl.ds(base, BLK)], tail_v, sem2)
        ci3 = pltpu.async_copy(types_hbm.at[pl.ds(base, BLK)], type_v, sem3)
        ci1.wait()
        ci2.wait()
        ci3.wait()
        cp1 = pltpu.async_copy(nf_hbm.at[tail_v], nfr_v, sem1)
        cp2 = pltpu.async_copy(rel_sh.at[type_v], relr_v, sem2)
        off = c * N
        for g in range(BLK // LANES):
            sl = pl.ds(g * LANES, LANES)
            tail_v[sl] = tail_v[sl] + off
        cp3 = pltpu.async_copy(dinvw_hbm.at[tail_v], wblk_v, sem3)
        cp1.wait()
        cp2.wait()
        cp3.wait()
        def ecomp(e, _):
            wj = wblk_v[e, pl.ds(0, LANES)]
            for ch in range(D // LANES):
                sl = pl.ds(ch * LANES, LANES)
                outr_v[e, sl] = wj * (nfr_v[e, sl] - relr_v[e, sl])
            return 0
        lax.fori_loop(0, BLK, ecomp, 0)
        pltpu.sync_copy(outr_v, acc_sh.at[head_v], add=True)
        return 0
    lax.fori_loop(0, NBLK, blk, 0)
    plsc.subcore_barrier()

    for k in range(CPT):
        idx = s + NSUB * k

        @pl.when(idx < NCHUNK)
        def _(idx=idx):
            r0 = idx * ZCH
            pltpu.sync_copy(acc_sh.at[pl.ds(r0, ZCH)],
                            acc_hbm.at[pl.ds(c * N + r0, ZCH)])
    plsc.subcore_barrier()


def _sc_pass(heads_f, tails_f, types_f, nf, relp):
    k = pl.kernel(
        _sc_body,
        out_type=(jax.ShapeDtypeStruct((2 * N, D), jnp.float32),
                  jax.ShapeDtypeStruct((2 * N, D), jnp.float32)),
        mesh=_sc_mesh,
        scratch_types=[
            pltpu.VMEM((BLK,), jnp.int32),
            pltpu.VMEM((BLK,), jnp.int32),
            pltpu.VMEM((BLK,), jnp.int32),
            pltpu.VMEM((BLK, D), jnp.float32),
            pltpu.VMEM((BLK, D), jnp.float32),
            pltpu.VMEM((BLK, D), jnp.float32),
            pltpu.VMEM((BLK, D), jnp.float32),
            pltpu.SemaphoreType.DMA,
            pltpu.SemaphoreType.DMA,
            pltpu.SemaphoreType.DMA,
            pltpu.VMEM_SHARED((N, D), jnp.float32),
            pltpu.VMEM_SHARED((RP, D), jnp.float32),
        ],
    )
    return k(heads_f, tails_f, types_f, nf, relp)


def _rel_body(rw_ref, basis_ref, loop_ref, wr_ref, rel_ref, out2_ref):
    rel_main = jnp.dot(rw_ref[...], basis_ref[...],
                       preferred_element_type=jnp.float32)
    rows = lax.broadcasted_iota(jnp.int32, (RP, D), 0)
    rel = jnp.where(rows == 2 * 237, loop_ref[0:1, :], rel_main)
    rel_ref[...] = rel
    out2_ref[...] = jnp.dot(rel, wr_ref[...], preferred_element_type=jnp.float32)


def _tc_rel(rw_pad, basis_pad, loop8, weight_rel):
    return pl.pallas_call(
        _rel_body,
        out_shape=(jax.ShapeDtypeStruct((RP, D), jnp.float32),
                   jax.ShapeDtypeStruct((RP, D), jnp.float32)),
    )(rw_pad, basis_pad, loop8, weight_rel)


def _final_body(accin_ref, accout_ref, din_ref, dout_ref, nf_ref, loop_ref,
                win_ref, wout_ref, wloop_ref, out_ref):
    a = jnp.dot(din_ref[...] * accin_ref[...], win_ref[...],
                preferred_element_type=jnp.float32)
    a += jnp.dot(dout_ref[...] * accout_ref[...], wout_ref[...],
                 preferred_element_type=jnp.float32)
    a += jnp.dot(nf_ref[...] - loop_ref[0:1, :], wloop_ref[...],
                 preferred_element_type=jnp.float32)
    out_ref[...] = jnp.tanh(a * (1.0 / 3.0))


def _tc_final(acc_in, acc_out, dinv_in, dinv_out, nf, loop8, w_in, w_out, w_loop):
    BR = 1000
    grid = (N // BR,)
    row_bs = pl.BlockSpec((BR, D), lambda i: (i, 0))
    col_bs = pl.BlockSpec((BR, 1), lambda i: (i, 0))
    full_bs = pl.BlockSpec((D, D), lambda i: (0, 0))
    loop_bs = pl.BlockSpec((8, D), lambda i: (0, 0))
    return pl.pallas_call(
        _final_body,
        grid=grid,
        in_specs=[row_bs, row_bs, col_bs, col_bs, row_bs, loop_bs,
                  full_bs, full_bs, full_bs],
        out_specs=row_bs,
        out_shape=jax.ShapeDtypeStruct((N, D), jnp.float32),
    )(acc_in, acc_out, dinv_in, dinv_out, nf, loop8, w_in, w_out, w_loop)


def kernel(nodes_features, edge_index, edge_type, basis_vector, rel_weight,
           weight_rel, loop_rel, w_in, w_out, w_loop):
    nf = nodes_features
    heads_f = edge_index[0]
    tails_f = edge_index[1]
    types_f = edge_type

    rw_pad = jnp.pad(rel_weight, ((0, RP - 474), (0, NBP - 50)))
    basis_pad = jnp.pad(basis_vector, ((0, NBP - 50), (0, 0)))
    loop8 = jnp.broadcast_to(loop_rel, (8, D))
    relp, out2p = _tc_rel(rw_pad, basis_pad, loop8, weight_rel)
    out_2 = out2p[:475]

    acc, dinvw = _sc_pass(heads_f, tails_f, types_f, nf, relp)
    dinv2 = dinvw[:, 0].reshape(2, N)

    out = _tc_final(acc[:N], acc[N:], dinv2[0][:, None], dinv2[1][:, None],
                    nf, loop8, w_in, w_out, w_loop)
    return (out, out_2)
